# baseline (speedup 1.0000x reference)
import functools

import jax
import jax.numpy as jnp
from jax import lax
from jax.experimental import pallas as pl
from jax.experimental.pallas import tpu as pltpu
from jax.experimental.pallas import tpu_sc as plsc

B, L, T, D = 8, 6, 512, 1024
LAST_LAYER = 6
MAX_ITERS = 6
N_ROWS = LAST_LAYER * MAX_ITERS
N_MEM = N_ROWS + 1
MSK_PAD = 48
LANES = 16
WPB = 4
ROWS_PER_W = N_ROWS // WPB
NG = 3
GR = ROWS_PER_W // NG


def _sc_rolling(lm2d, valid2d, layer_embedding, non2d):
    mesh = plsc.VectorSubcoreMesh(core_axis_name="c", subcore_axis_name="s")

    @functools.partial(
        pl.kernel,
        out_type=[
            jax.ShapeDtypeStruct((N_MEM, B, D), jnp.float32),
            jax.ShapeDtypeStruct((B, MSK_PAD), jnp.int32),
        ],
        mesh=mesh,
        compiler_params=pltpu.CompilerParams(needs_layout_passes=False),
        scratch_types=[
            pltpu.VMEM((1, T), jnp.int32),
            pltpu.VMEM((L, D), jnp.float32),
            [pltpu.VMEM((GR, D), jnp.float32) for _ in range(NG)],
            [pltpu.VMEM((LANES,), jnp.int32) for _ in range(NG)],
            pltpu.VMEM((1, D), jnp.float32),
            pltpu.VMEM((1, MSK_PAD), jnp.int32),
            pltpu.SemaphoreType.DMA,
            pltpu.SemaphoreType.DMA,
            [pltpu.SemaphoreType.DMA for _ in range(NG)],
            pltpu.SemaphoreType.DMA,
        ],
    )
    def k(lm_hbm, valid_hbm, layer_hbm, non_hbm, mem_hbm, msk_hbm,
          valid_v, layer_v, rows_g, idx_g, non_v, msk_v,
          sem_l, sem_v, sem_g, sem_s):
        wid = lax.axis_index("s") * mesh.num_cores + lax.axis_index("c")
        b = wid // WPB
        j = wid % WPB

        layer_cp = pltpu.async_copy(layer_hbm, layer_v, sem_l)
        valid_cp = pltpu.async_copy(valid_hbm.at[pl.ds(b, 1)], valid_v, sem_v)

        lane = lax.iota(jnp.int32, LANES)
        valid_cp.wait()

        def chunk_min(i, cur):
            c = valid_v[0, pl.ds(i * LANES, LANES)]
            return jnp.minimum(cur, jnp.where(c != 0, lane + i * LANES, T))

        minv = lax.fori_loop(0, T // LANES, chunk_min,
                             jnp.full((LANES,), T, jnp.int32), unroll=8)
        first = jnp.min(minv)
        first = jnp.where(first >= T, 0, first)

        gathers = []
        for g in range(NG):
            rg = j * ROWS_PER_W + jnp.minimum(g * GR + lane, ROWS_PER_W - 1)
            lg = rg % LAST_LAYER
            tg = jnp.minimum(first + rg // LAST_LAYER, T - 1)
            idx_g[g][...] = (b * L + lg) * T + tg
            gathers.append(pltpu.async_copy(
                lm_hbm.at[idx_g[g].at[pl.ds(0, GR)]], rows_g[g], sem_g[g]))

        @pl.when(j == 0)
        def _():
            pltpu.sync_copy(non_hbm, non_v)
            pltpu.sync_copy(non_v, mem_hbm.at[N_MEM - 1, pl.ds(b, 1)])
            for i in range(MSK_PAD // LANES):
                p = lane + i * LANES
                pt = jnp.minimum(first + p // LAST_LAYER, T - 1)
                v = plsc.load_gather(valid_v, [jnp.zeros((LANES,), jnp.int32),
                                               pt])
                msk_v[0, pl.ds(i * LANES, LANES)] = jnp.where(
                    (v == 0) & (p < N_ROWS), 1, 0)
            pltpu.sync_copy(msk_v, msk_hbm.at[pl.ds(b, 1)])

        layer_cp.wait()
        CH = D // LANES
        stores = []
        for g in range(NG):
            gathers[g].wait()
            buf = rows_g[g]

            def add_g(i, _, g=g, buf=buf):
                kk = i // CH
                sl = pl.ds((i % CH) * LANES, LANES)
                lk = (j * ROWS_PER_W + g * GR + kk) % LAST_LAYER
                buf[kk, sl] = buf[kk, sl] + layer_v[lk, sl]
                return 0

            lax.fori_loop(0, GR * CH, add_g, 0, unroll=8)
            for m in range(GR):
                rk = j * ROWS_PER_W + g * GR + m
                stores.append(pltpu.async_copy(
                    buf.at[pl.ds(m, 1)], mem_hbm.at[rk, pl.ds(b, 1)], sem_s))
        for cp in stores:
            cp.wait()

    return k(lm2d, valid2d, layer_embedding, non2d)


def kernel(lm_emb, lm_emb_valid, layer_embedding, nonsense_embedding):
    mem_t, msk_i32 = _sc_rolling(
        lm_emb.reshape(B * L * T, D),
        lm_emb_valid.astype(jnp.int32),
        layer_embedding,
        nonsense_embedding)
    mem = jnp.transpose(mem_t, (1, 0, 2))
    msk = msk_i32[:, :N_MEM] != 0
    return mem, msk


# speedup vs baseline: 1.0108x; 1.0108x over previous
import functools

import jax
import jax.numpy as jnp
from jax import lax
from jax.experimental import pallas as pl
from jax.experimental.pallas import tpu as pltpu
from jax.experimental.pallas import tpu_sc as plsc

B, L, T, D = 8, 6, 512, 1024
LAST_LAYER = 6
MAX_ITERS = 6
N_ROWS = LAST_LAYER * MAX_ITERS
N_MEM = N_ROWS + 1
MSK_PAD = 48
LANES = 16
WPB = 4
ROWS_PER_W = N_ROWS // WPB


def _sc_rolling(lm2d, valid2d, layer_embedding, non2d):
    mesh = plsc.VectorSubcoreMesh(core_axis_name="c", subcore_axis_name="s")

    @functools.partial(
        pl.kernel,
        out_type=[
            jax.ShapeDtypeStruct((N_MEM, B, D), jnp.float32),
            jax.ShapeDtypeStruct((B, MSK_PAD), jnp.int32),
        ],
        mesh=mesh,
        compiler_params=pltpu.CompilerParams(needs_layout_passes=False),
        scratch_types=[
            pltpu.VMEM((1, T), jnp.int32),
            pltpu.VMEM((L, D), jnp.float32),
            pltpu.VMEM((8, D), jnp.float32),
            pltpu.VMEM((1, D), jnp.float32),
            pltpu.VMEM((LANES,), jnp.int32),
            pltpu.VMEM((1, D), jnp.float32),
            pltpu.VMEM((1, MSK_PAD), jnp.int32),
            pltpu.SemaphoreType.DMA,
            pltpu.SemaphoreType.DMA,
            pltpu.SemaphoreType.DMA,
            pltpu.SemaphoreType.DMA,
            pltpu.SemaphoreType.DMA,
        ],
    )
    def k(lm_hbm, valid_hbm, layer_hbm, non_hbm, mem_hbm, msk_hbm,
          valid_v, layer_v, rows8_v, rows1_v, gidx_v, non_v, msk_v,
          sem_l, sem_v, sem_g, sem_h, sem_s):
        wid = lax.axis_index("s") * mesh.num_cores + lax.axis_index("c")
        b = wid // WPB
        j = wid % WPB

        layer_cp = pltpu.async_copy(layer_hbm, layer_v, sem_l)
        valid_cp = pltpu.async_copy(valid_hbm.at[pl.ds(b, 1)], valid_v, sem_v)

        lane = lax.iota(jnp.int32, LANES)
        valid_cp.wait()

        def chunk_min(i, cur):
            c = valid_v[0, pl.ds(i * LANES, LANES)]
            return jnp.minimum(cur, jnp.where(c != 0, lane + i * LANES, T))

        minv = lax.fori_loop(0, T // LANES, chunk_min,
                             jnp.full((LANES,), T, jnp.int32), unroll=8)
        first = jnp.min(minv)
        first = jnp.where(first >= T, 0, first)

        r = j * ROWS_PER_W + jnp.minimum(lane, ROWS_PER_W - 1)
        l = r % LAST_LAYER
        t = jnp.minimum(first + r // LAST_LAYER, T - 1)
        gidx_v[...] = (b * L + l) * T + t
        g8_cp = pltpu.async_copy(lm_hbm.at[gidx_v.at[pl.ds(0, 8)]],
                                 rows8_v, sem_g)
        g1_cp = pltpu.async_copy(lm_hbm.at[gidx_v.at[pl.ds(8, 1)]],
                                 rows1_v, sem_h)

        @pl.when(j == 0)
        def _():
            pltpu.sync_copy(non_hbm, non_v)
            pltpu.sync_copy(non_v, mem_hbm.at[N_MEM - 1, pl.ds(b, 1)])
            for i in range(MSK_PAD // LANES):
                p = lane + i * LANES
                pt = jnp.minimum(first + p // LAST_LAYER, T - 1)
                v = plsc.load_gather(valid_v, [jnp.zeros((LANES,), jnp.int32),
                                               pt])
                msk_v[0, pl.ds(i * LANES, LANES)] = jnp.where(
                    (v == 0) & (p < N_ROWS), 1, 0)
            pltpu.sync_copy(msk_v, msk_hbm.at[pl.ds(b, 1)])

        layer_cp.wait()
        g8_cp.wait()

        CH = D // LANES

        def add8(i, _):
            kk = i // CH
            sl = pl.ds((i % CH) * LANES, LANES)
            lk = (j * ROWS_PER_W + kk) % LAST_LAYER
            rows8_v[kk, sl] = rows8_v[kk, sl] + layer_v[lk, sl]
            return 0

        lax.fori_loop(0, 8 * CH, add8, 0, unroll=8)

        stores = []
        for kk in range(8):
            rk = j * ROWS_PER_W + kk
            stores.append(pltpu.async_copy(
                rows8_v.at[pl.ds(kk, 1)], mem_hbm.at[rk, pl.ds(b, 1)], sem_s))

        g1_cp.wait()
        l8 = (j * ROWS_PER_W + 8) % LAST_LAYER

        def add1(i, _):
            sl = pl.ds(i * LANES, LANES)
            rows1_v[0, sl] = rows1_v[0, sl] + layer_v[l8, sl]
            return 0

        lax.fori_loop(0, CH, add1, 0, unroll=4)
        stores.append(pltpu.async_copy(
            rows1_v, mem_hbm.at[j * ROWS_PER_W + 8, pl.ds(b, 1)], sem_s))
        for cp in stores:
            cp.wait()

    return k(lm2d, valid2d, layer_embedding, non2d)


def kernel(lm_emb, lm_emb_valid, layer_embedding, nonsense_embedding):
    mem_t, msk_i32 = _sc_rolling(
        lm_emb.reshape(B * L * T, D),
        lm_emb_valid.astype(jnp.int32),
        layer_embedding,
        nonsense_embedding)
    mem = jnp.transpose(mem_t, (1, 0, 2))
    msk = msk_i32[:, :N_MEM] != 0
    return mem, msk
